# baseline (speedup 1.0000x reference)
import functools

import jax
import jax.numpy as jnp
from jax import lax
from jax.experimental import pallas as pl
from jax.experimental.pallas import tpu as pltpu
from jax.experimental.pallas import tpu_sc as plsc

N = 10000
E = 100000
D = 32
DE = 16
DH = 128
STEPS = 2

NC = 2
NS = 16
NW = NC * NS
GCH = 128
NCH = 13
EPW = NCH * GCH
E_HALF = NW * EPW
E_PAD = 2 * E_HALF
N_ACC = 10112
RPS = N_ACC // NS
TE = 1024


@functools.lru_cache(maxsize=1)
def _sc_kernels():
    mesh = plsc.VectorSubcoreMesh(core_axis_name="c", subcore_axis_name="s")

    @functools.partial(
        pl.kernel,
        mesh=mesh,
        out_type=jax.ShapeDtypeStruct((E_HALF, D), jnp.float32),
        scratch_types=[
            pltpu.VMEM((NCH, GCH), jnp.int32),
            pltpu.VMEM((EPW, D), jnp.float32),
            pltpu.SemaphoreType.DMA,
        ],
        compiler_params=pltpu.CompilerParams(use_tc_tiling_on_sc=False),
    )
    def _sc_gather(nodes_hbm, src_hbm, hsrc_hbm, idx_v, rows_v, sem):
        c = lax.axis_index("c")
        s = lax.axis_index("s")
        wid = s * NC + c
        base = wid * EPW
        pltpu.sync_copy(src_hbm.at[wid], idx_v)

        def _issue(j, carry):
            pltpu.async_copy(nodes_hbm.at[idx_v.at[j]],
                             rows_v.at[pl.ds(j * GCH, GCH)], sem)
            return carry

        lax.fori_loop(0, NCH, _issue, 0)

        def _drain(j, carry):
            pltpu.make_async_copy(nodes_hbm.at[idx_v.at[0]],
                                  rows_v.at[pl.ds(0, GCH)], sem).wait()
            return carry

        lax.fori_loop(0, NCH, _drain, 0)
        pltpu.sync_copy(rows_v, hsrc_hbm.at[pl.ds(base, EPW)])

    @functools.partial(
        pl.kernel,
        mesh=mesh,
        out_type=jax.ShapeDtypeStruct((NC, N_ACC, D), jnp.float32),
        scratch_types=[
            pltpu.VMEM((NCH, GCH), jnp.int32),
            pltpu.VMEM((EPW, D), jnp.float32),
            pltpu.VMEM_SHARED((N_ACC, D), jnp.float32),
            pltpu.SemaphoreType.DMA,
        ],
        compiler_params=pltpu.CompilerParams(use_tc_tiling_on_sc=False),
    )
    def _sc_scatter(msg_hbm, dst_hbm, zeros_hbm, part_hbm,
                    idx_v, rows_v, acc_sh, sem):
        c = lax.axis_index("c")
        s = lax.axis_index("s")
        wid = c * NS + s
        base = wid * EPW

        pltpu.sync_copy(zeros_hbm.at[pl.ds(s * RPS, RPS)],
                        acc_sh.at[pl.ds(s * RPS, RPS)])
        plsc.subcore_barrier()

        pltpu.sync_copy(dst_hbm.at[wid], idx_v)
        pltpu.sync_copy(msg_hbm.at[pl.ds(base, EPW)], rows_v)

        def _scat(j, carry):
            pltpu.sync_copy(rows_v.at[pl.ds(j * GCH, GCH)],
                            acc_sh.at[idx_v.at[j]], add=True)
            return carry

        lax.fori_loop(0, NCH, _scat, 0)
        plsc.subcore_barrier()

        pltpu.sync_copy(acc_sh.at[pl.ds(s * RPS, RPS)],
                        part_hbm.at[c, pl.ds(s * RPS, RPS)])

    return _sc_gather, _sc_scatter


def _msg_body(ef_ref, hs_ref, we1_ref, be1_ref, we2_ref, be2_ref, p_ref,
              msg_ref):
    henc = jnp.maximum(
        jnp.dot(ef_ref[...], we1_ref[...], preferred_element_type=jnp.float32)
        + be1_ref[...], 0.0)
    ewt = jnp.dot(henc, we2_ref[...],
                  preferred_element_type=jnp.float32) + be2_ref[...]
    hrep = jnp.dot(hs_ref[...], p_ref[...], preferred_element_type=jnp.float32)
    prod = hrep * ewt
    prod = prod[:, :512] + prod[:, 512:]
    prod = prod[:, :256] + prod[:, 256:]
    prod = prod[:, :128] + prod[:, 128:]
    prod = prod[:, :64] + prod[:, 64:]
    msg_ref[...] = prod[:, :32] + prod[:, 32:]


def _msg_kernel(e_feat_p, h_src, We1, be1, We2, be2, P):
    grid = (E_HALF // TE,)
    return pl.pallas_call(
        _msg_body,
        grid=grid,
        in_specs=[
            pl.BlockSpec((TE, DE), lambda i: (i, 0)),
            pl.BlockSpec((TE, D), lambda i: (i, 0)),
            pl.BlockSpec((DE, DH), lambda i: (0, 0)),
            pl.BlockSpec((1, DH), lambda i: (0, 0)),
            pl.BlockSpec((DH, D * D), lambda i: (0, 0)),
            pl.BlockSpec((1, D * D), lambda i: (0, 0)),
            pl.BlockSpec((D, D * D), lambda i: (0, 0)),
        ],
        out_specs=pl.BlockSpec((TE, D), lambda i: (i, 0)),
        out_shape=jax.ShapeDtypeStruct((E_HALF, D), jnp.float32),
    )(e_feat_p, h_src, We1, be1, We2, be2, P)


def _prologue_body(nf_ref, w0_ref, b0_ref, out_ref):
    out_ref[...] = jnp.maximum(
        jnp.dot(nf_ref[...], w0_ref[...], preferred_element_type=jnp.float32)
        + b0_ref[...], 0.0)


def _prologue(n_feat, W0, b0):
    return pl.pallas_call(
        _prologue_body,
        out_shape=jax.ShapeDtypeStruct((N, D), jnp.float32),
    )(n_feat, W0, b0)


def _update_body(pa_ref, pb_ref, out_ref, cb_ref, wm1_ref, wm2_ref, bm_ref,
                 o_ref):
    out = out_ref[...]
    neigh = (pa_ref[0, :N, :] + pa_ref[1, :N, :]
             + pb_ref[0, :N, :] + pb_ref[1, :N, :])
    m = jnp.maximum(neigh + out + cb_ref[...], 0.0)
    o_ref[...] = (jnp.dot(m, wm1_ref[...], preferred_element_type=jnp.float32)
                  + jnp.dot(out, wm2_ref[...],
                            preferred_element_type=jnp.float32)
                  + bm_ref[...])


def _final_body(pa_ref, pb_ref, out_ref, cb_ref, wm1_ref, wm2_ref, bm_ref,
                init_ref, o_ref):
    out = out_ref[...]
    neigh = (pa_ref[0, :N, :] + pa_ref[1, :N, :]
             + pb_ref[0, :N, :] + pb_ref[1, :N, :])
    m = jnp.maximum(neigh + out + cb_ref[...], 0.0)
    o_ref[...] = (jnp.dot(m, wm1_ref[...], preferred_element_type=jnp.float32)
                  + jnp.dot(out, wm2_ref[...],
                            preferred_element_type=jnp.float32)
                  + bm_ref[...] + init_ref[...])


def _update(pa, pb, out, cb, wm1, wm2, bm):
    return pl.pallas_call(
        _update_body,
        out_shape=jax.ShapeDtypeStruct((N, D), jnp.float32),
    )(pa, pb, out, cb, wm1, wm2, bm)


def _final(pa, pb, out, cb, wm1, wm2, bm, init):
    return pl.pallas_call(
        _final_body,
        out_shape=jax.ShapeDtypeStruct((N, D), jnp.float32),
    )(pa, pb, out, cb, wm1, wm2, bm, init)


def kernel(edge_index, n_feat, e_feat, W0, b0, We1, be1, We2, be2, conv_bias,
           Wm, bm):
    src = edge_index[0]
    dst = edge_index[1]
    npad = E_PAD - E
    fill = (jnp.arange(npad, dtype=jnp.int32) % (N_ACC - N))
    src_p = jnp.concatenate([src, fill]).reshape(2, NW, NCH, GCH)
    dst_p = jnp.concatenate([dst, N + fill]).reshape(2, NW, NCH, GCH)
    e_feat_p = jnp.concatenate(
        [e_feat, jnp.zeros((npad, DE), jnp.float32)], axis=0)
    zeros_acc = jnp.zeros((N_ACC, D), jnp.float32)
    be1_2 = be1.reshape(1, DH)
    be2_2 = be2.reshape(1, D * D)
    cb_2 = conv_bias.reshape(1, D)
    bm_2 = bm.reshape(1, D)
    b0_2 = b0.reshape(1, D)
    wm1 = Wm[:D]
    wm2 = Wm[D:]
    P = jnp.kron(jnp.eye(D, dtype=jnp.float32), jnp.ones((1, D), jnp.float32))

    ef_halves = (e_feat_p[:E_HALF], e_feat_p[E_HALF:])
    sc_gather, sc_scatter = _sc_kernels()
    out = _prologue(n_feat, W0, b0_2)
    for step in range(STEPS):
        hs = [sc_gather(out, src_p[h]) for h in range(2)]
        parts = []
        for h in range(2):
            msg = _msg_kernel(ef_halves[h], hs[h], We1, be1_2, We2, be2_2, P)
            parts.append(sc_scatter(msg, dst_p[h], zeros_acc))
        if step == STEPS - 1:
            out = _final(parts[0], parts[1], out, cb_2, wm1, wm2, bm_2, n_feat)
        else:
            out = _update(parts[0], parts[1], out, cb_2, wm1, wm2, bm_2)
    return out


# speedup vs baseline: 1.0685x; 1.0685x over previous
import functools

import jax
import jax.numpy as jnp
from jax import lax
from jax.experimental import pallas as pl
from jax.experimental.pallas import tpu as pltpu
from jax.experimental.pallas import tpu_sc as plsc

N = 10000
E = 100000
D = 32
DE = 16
DH = 128
STEPS = 2

NC = 2
NS = 16
NW = NC * NS
GCH = 128
NCH = 25
EPW = NCH * GCH
E_PAD = NW * EPW
N_ACC = 10112
RPS = N_ACC // NS
TE = 1024


@functools.lru_cache(maxsize=1)
def _sc_kernels():
    mesh = plsc.VectorSubcoreMesh(core_axis_name="c", subcore_axis_name="s")

    @functools.partial(
        pl.kernel,
        mesh=mesh,
        out_type=jax.ShapeDtypeStruct((E_PAD, D), jnp.float32),
        scratch_types=[
            pltpu.VMEM((NCH, GCH), jnp.int32),
            pltpu.VMEM((EPW, D), jnp.float32),
            pltpu.SemaphoreType.DMA,
        ],
        compiler_params=pltpu.CompilerParams(use_tc_tiling_on_sc=False),
    )
    def _sc_gather(nodes_hbm, src_hbm, hsrc_hbm, idx_v, rows_v, sem):
        c = lax.axis_index("c")
        s = lax.axis_index("s")
        wid = s * NC + c
        base = wid * EPW
        pltpu.sync_copy(src_hbm.at[wid], idx_v)

        def _issue(j, carry):
            pltpu.async_copy(nodes_hbm.at[idx_v.at[j]],
                             rows_v.at[pl.ds(j * GCH, GCH)], sem)
            return carry

        lax.fori_loop(0, NCH, _issue, 0)

        def _drain(j, carry):
            pltpu.make_async_copy(nodes_hbm.at[idx_v.at[0]],
                                  rows_v.at[pl.ds(0, GCH)], sem).wait()
            return carry

        lax.fori_loop(0, NCH, _drain, 0)
        pltpu.sync_copy(rows_v, hsrc_hbm.at[pl.ds(base, EPW)])

    @functools.partial(
        pl.kernel,
        mesh=mesh,
        out_type=jax.ShapeDtypeStruct((NC, N_ACC, D), jnp.float32),
        scratch_types=[
            pltpu.VMEM((NCH, GCH), jnp.int32),
            pltpu.VMEM((EPW, D), jnp.float32),
            pltpu.VMEM_SHARED((N_ACC, D), jnp.float32),
            pltpu.SemaphoreType.DMA,
        ],
        compiler_params=pltpu.CompilerParams(use_tc_tiling_on_sc=False),
    )
    def _sc_scatter(msg_hbm, dst_hbm, zeros_hbm, part_hbm,
                    idx_v, rows_v, acc_sh, sem):
        c = lax.axis_index("c")
        s = lax.axis_index("s")
        wid = c * NS + s
        base = wid * EPW

        pltpu.sync_copy(zeros_hbm.at[pl.ds(s * RPS, RPS)],
                        acc_sh.at[pl.ds(s * RPS, RPS)])
        plsc.subcore_barrier()

        pltpu.sync_copy(dst_hbm.at[wid], idx_v)
        pltpu.sync_copy(msg_hbm.at[pl.ds(base, EPW)], rows_v)

        def _scat(j, carry):
            pltpu.sync_copy(rows_v.at[pl.ds(j * GCH, GCH)],
                            acc_sh.at[idx_v.at[j]], add=True)
            return carry

        lax.fori_loop(0, NCH, _scat, 0)
        plsc.subcore_barrier()

        pltpu.sync_copy(acc_sh.at[pl.ds(s * RPS, RPS)],
                        part_hbm.at[c, pl.ds(s * RPS, RPS)])

    return _sc_gather, _sc_scatter


def _msg_body(ef_ref, hs_ref, we1_ref, be1_ref, we2_ref, be2_ref, p_ref,
              msg_ref):
    henc = jnp.maximum(
        jnp.dot(ef_ref[...], we1_ref[...], preferred_element_type=jnp.float32)
        + be1_ref[...], 0.0)
    ewt = jnp.dot(henc, we2_ref[...],
                  preferred_element_type=jnp.float32) + be2_ref[...]
    hrep = jnp.dot(hs_ref[...], p_ref[...], preferred_element_type=jnp.float32)
    prod = hrep * ewt
    prod = prod[:, :512] + prod[:, 512:]
    prod = prod[:, :256] + prod[:, 256:]
    prod = prod[:, :128] + prod[:, 128:]
    prod = prod[:, :64] + prod[:, 64:]
    msg_ref[...] = prod[:, :32] + prod[:, 32:]


def _msg_kernel(e_feat_p, h_src, We1, be1, We2, be2, P):
    grid = (E_PAD // TE,)
    return pl.pallas_call(
        _msg_body,
        grid=grid,
        in_specs=[
            pl.BlockSpec((TE, DE), lambda i: (i, 0)),
            pl.BlockSpec((TE, D), lambda i: (i, 0)),
            pl.BlockSpec((DE, DH), lambda i: (0, 0)),
            pl.BlockSpec((1, DH), lambda i: (0, 0)),
            pl.BlockSpec((DH, D * D), lambda i: (0, 0)),
            pl.BlockSpec((1, D * D), lambda i: (0, 0)),
            pl.BlockSpec((D, D * D), lambda i: (0, 0)),
        ],
        out_specs=pl.BlockSpec((TE, D), lambda i: (i, 0)),
        out_shape=jax.ShapeDtypeStruct((E_PAD, D), jnp.float32),
    )(e_feat_p, h_src, We1, be1, We2, be2, P)


def _prologue_body(nf_ref, w0_ref, b0_ref, out_ref):
    out_ref[...] = jnp.maximum(
        jnp.dot(nf_ref[...], w0_ref[...], preferred_element_type=jnp.float32)
        + b0_ref[...], 0.0)


def _prologue(n_feat, W0, b0):
    return pl.pallas_call(
        _prologue_body,
        out_shape=jax.ShapeDtypeStruct((N, D), jnp.float32),
    )(n_feat, W0, b0)


def _update_body(pa_ref, out_ref, cb_ref, wm1_ref, wm2_ref, bm_ref,
                 o_ref):
    out = out_ref[...]
    neigh = pa_ref[0, :N, :] + pa_ref[1, :N, :]
    m = jnp.maximum(neigh + out + cb_ref[...], 0.0)
    o_ref[...] = (jnp.dot(m, wm1_ref[...], preferred_element_type=jnp.float32)
                  + jnp.dot(out, wm2_ref[...],
                            preferred_element_type=jnp.float32)
                  + bm_ref[...])


def _final_body(pa_ref, out_ref, cb_ref, wm1_ref, wm2_ref, bm_ref,
                init_ref, o_ref):
    out = out_ref[...]
    neigh = pa_ref[0, :N, :] + pa_ref[1, :N, :]
    m = jnp.maximum(neigh + out + cb_ref[...], 0.0)
    o_ref[...] = (jnp.dot(m, wm1_ref[...], preferred_element_type=jnp.float32)
                  + jnp.dot(out, wm2_ref[...],
                            preferred_element_type=jnp.float32)
                  + bm_ref[...] + init_ref[...])


def _update(pa, out, cb, wm1, wm2, bm):
    return pl.pallas_call(
        _update_body,
        out_shape=jax.ShapeDtypeStruct((N, D), jnp.float32),
    )(pa, out, cb, wm1, wm2, bm)


def _final(pa, out, cb, wm1, wm2, bm, init):
    return pl.pallas_call(
        _final_body,
        out_shape=jax.ShapeDtypeStruct((N, D), jnp.float32),
    )(pa, out, cb, wm1, wm2, bm, init)


def kernel(edge_index, n_feat, e_feat, W0, b0, We1, be1, We2, be2, conv_bias,
           Wm, bm):
    src = edge_index[0]
    dst = edge_index[1]
    npad = E_PAD - E
    fill = (jnp.arange(npad, dtype=jnp.int32) % (N_ACC - N))
    src_p = jnp.concatenate([src, fill]).reshape(NW, NCH, GCH)
    dst_p = jnp.concatenate([dst, N + fill]).reshape(NW, NCH, GCH)
    e_feat_p = jnp.concatenate(
        [e_feat, jnp.zeros((npad, DE), jnp.float32)], axis=0)
    zeros_acc = jnp.zeros((N_ACC, D), jnp.float32)
    be1_2 = be1.reshape(1, DH)
    be2_2 = be2.reshape(1, D * D)
    cb_2 = conv_bias.reshape(1, D)
    bm_2 = bm.reshape(1, D)
    b0_2 = b0.reshape(1, D)
    wm1 = Wm[:D]
    wm2 = Wm[D:]
    P = jnp.kron(jnp.eye(D, dtype=jnp.float32), jnp.ones((1, D), jnp.float32))

    sc_gather, sc_scatter = _sc_kernels()
    out = _prologue(n_feat, W0, b0_2)
    for step in range(STEPS):
        h_src = sc_gather(out, src_p)
        msg = _msg_kernel(e_feat_p, h_src, We1, be1_2, We2, be2_2, P)
        parts = sc_scatter(msg, dst_p, zeros_acc)
        if step == STEPS - 1:
            out = _final(parts, out, cb_2, wm1, wm2, bm_2, n_feat)
        else:
            out = _update(parts, out, cb_2, wm1, wm2, bm_2)
    return out


# speedup vs baseline: 1.0700x; 1.0014x over previous
import functools

import jax
import jax.numpy as jnp
from jax import lax
from jax.experimental import pallas as pl
from jax.experimental.pallas import tpu as pltpu
from jax.experimental.pallas import tpu_sc as plsc

N = 10000
E = 100000
D = 32
DE = 16
DH = 128
STEPS = 2

NC = 2
NS = 16
NW = NC * NS
GCH = 128
NCH = 25
EPW = NCH * GCH
E_PAD = NW * EPW
N_ACC = 10112
RPS = N_ACC // NS
TE = 1024


@functools.lru_cache(maxsize=1)
def _sc_kernels():
    mesh = plsc.VectorSubcoreMesh(core_axis_name="c", subcore_axis_name="s")

    @functools.partial(
        pl.kernel,
        mesh=mesh,
        out_type=jax.ShapeDtypeStruct((E_PAD, D), jnp.float32),
        scratch_types=[
            pltpu.VMEM((NCH, GCH), jnp.int32),
            pltpu.VMEM((EPW, D), jnp.float32),
            pltpu.VMEM_SHARED((N_ACC, D), jnp.float32),
            pltpu.SemaphoreType.DMA,
        ],
        compiler_params=pltpu.CompilerParams(use_tc_tiling_on_sc=False),
    )
    def _sc_gather(nodes_hbm, src_hbm, hsrc_hbm, idx_v, rows_v, nodes_sh, sem):
        c = lax.axis_index("c")
        s = lax.axis_index("s")
        wid = s * NC + c
        base = wid * EPW
        pltpu.sync_copy(nodes_hbm.at[pl.ds(s * RPS, RPS)],
                        nodes_sh.at[pl.ds(s * RPS, RPS)])
        pltpu.sync_copy(src_hbm.at[wid], idx_v)
        plsc.subcore_barrier()

        def _issue(j, carry):
            pltpu.async_copy(nodes_sh.at[idx_v.at[j]],
                             rows_v.at[pl.ds(j * GCH, GCH)], sem)
            return carry

        lax.fori_loop(0, NCH, _issue, 0)

        def _drain(j, carry):
            pltpu.make_async_copy(hsrc_hbm.at[pl.ds(0, GCH)],
                                  rows_v.at[pl.ds(0, GCH)], sem).wait()
            return carry

        lax.fori_loop(0, NCH, _drain, 0)
        pltpu.sync_copy(rows_v, hsrc_hbm.at[pl.ds(base, EPW)])

    @functools.partial(
        pl.kernel,
        mesh=mesh,
        out_type=jax.ShapeDtypeStruct((NC, N_ACC, D), jnp.float32),
        scratch_types=[
            pltpu.VMEM((NCH, GCH), jnp.int32),
            pltpu.VMEM((EPW, D), jnp.float32),
            pltpu.VMEM_SHARED((N_ACC, D), jnp.float32),
            pltpu.SemaphoreType.DMA,
        ],
        compiler_params=pltpu.CompilerParams(use_tc_tiling_on_sc=False),
    )
    def _sc_scatter(msg_hbm, dst_hbm, zeros_hbm, part_hbm,
                    idx_v, rows_v, acc_sh, sem):
        c = lax.axis_index("c")
        s = lax.axis_index("s")
        wid = c * NS + s
        base = wid * EPW

        pltpu.sync_copy(zeros_hbm.at[pl.ds(s * RPS, RPS)],
                        acc_sh.at[pl.ds(s * RPS, RPS)])
        plsc.subcore_barrier()

        pltpu.sync_copy(dst_hbm.at[wid], idx_v)
        pltpu.sync_copy(msg_hbm.at[pl.ds(base, EPW)], rows_v)

        def _scat(j, carry):
            pltpu.sync_copy(rows_v.at[pl.ds(j * GCH, GCH)],
                            acc_sh.at[idx_v.at[j]], add=True)
            return carry

        lax.fori_loop(0, NCH, _scat, 0)
        plsc.subcore_barrier()

        pltpu.sync_copy(acc_sh.at[pl.ds(s * RPS, RPS)],
                        part_hbm.at[c, pl.ds(s * RPS, RPS)])

    return _sc_gather, _sc_scatter


def _msg_body(ef_ref, hs_ref, we1_ref, be1_ref, wc_ref, be2_ref, msg_ref):
    henc = jnp.maximum(
        jnp.dot(ef_ref[...], we1_ref[...], preferred_element_type=jnp.float32)
        + be1_ref[...], 0.0)
    comb = jnp.dot(jnp.concatenate([henc, hs_ref[...]], axis=1), wc_ref[...],
                   preferred_element_type=jnp.float32)
    ewt = comb[:, :D * D] + be2_ref[...]
    hrep = comb[:, D * D:]
    prod = (hrep[:, :512] * ewt[:, :512] + hrep[:, 512:] * ewt[:, 512:])
    prod = prod[:, :256] + prod[:, 256:]
    prod = prod[:, :128] + prod[:, 128:]
    prod = prod[:, :64] + prod[:, 64:]
    msg_ref[...] = prod[:, :32] + prod[:, 32:]


def _msg_kernel(e_feat_p, h_src, We1, be1, Wc, be2):
    grid = (E_PAD // TE,)
    return pl.pallas_call(
        _msg_body,
        grid=grid,
        in_specs=[
            pl.BlockSpec((TE, DE), lambda i: (i, 0)),
            pl.BlockSpec((TE, D), lambda i: (i, 0)),
            pl.BlockSpec((DE, DH), lambda i: (0, 0)),
            pl.BlockSpec((1, DH), lambda i: (0, 0)),
            pl.BlockSpec((DH + D, 2 * D * D), lambda i: (0, 0)),
            pl.BlockSpec((1, D * D), lambda i: (0, 0)),
        ],
        out_specs=pl.BlockSpec((TE, D), lambda i: (i, 0)),
        out_shape=jax.ShapeDtypeStruct((E_PAD, D), jnp.float32),
    )(e_feat_p, h_src, We1, be1, Wc, be2)


def _prologue_body(nf_ref, w0_ref, b0_ref, out_ref):
    out_ref[...] = jnp.maximum(
        jnp.dot(nf_ref[...], w0_ref[...], preferred_element_type=jnp.float32)
        + b0_ref[...], 0.0)


def _prologue(n_feat_acc, W0, b0):
    return pl.pallas_call(
        _prologue_body,
        out_shape=jax.ShapeDtypeStruct((N_ACC, D), jnp.float32),
    )(n_feat_acc, W0, b0)


def _update_body(pa_ref, out_ref, cb_ref, wm1_ref, wm2_ref, bm_ref,
                 o_ref):
    out = out_ref[...]
    neigh = pa_ref[0] + pa_ref[1]
    m = jnp.maximum(neigh + out + cb_ref[...], 0.0)
    o_ref[...] = (jnp.dot(m, wm1_ref[...], preferred_element_type=jnp.float32)
                  + jnp.dot(out, wm2_ref[...],
                            preferred_element_type=jnp.float32)
                  + bm_ref[...])


def _final_body(pa_ref, out_ref, cb_ref, wm1_ref, wm2_ref, bm_ref,
                init_ref, o_ref):
    out = out_ref[:N, :]
    neigh = pa_ref[0, :N, :] + pa_ref[1, :N, :]
    m = jnp.maximum(neigh + out + cb_ref[...], 0.0)
    o_ref[...] = (jnp.dot(m, wm1_ref[...], preferred_element_type=jnp.float32)
                  + jnp.dot(out, wm2_ref[...],
                            preferred_element_type=jnp.float32)
                  + bm_ref[...] + init_ref[...])


def _update(pa, out, cb, wm1, wm2, bm):
    return pl.pallas_call(
        _update_body,
        out_shape=jax.ShapeDtypeStruct((N_ACC, D), jnp.float32),
    )(pa, out, cb, wm1, wm2, bm)


def _final(pa, out, cb, wm1, wm2, bm, init):
    return pl.pallas_call(
        _final_body,
        out_shape=jax.ShapeDtypeStruct((N, D), jnp.float32),
    )(pa, out, cb, wm1, wm2, bm, init)


def kernel(edge_index, n_feat, e_feat, W0, b0, We1, be1, We2, be2, conv_bias,
           Wm, bm):
    src = edge_index[0]
    dst = edge_index[1]
    npad = E_PAD - E
    fill = (jnp.arange(npad, dtype=jnp.int32) % (N_ACC - N))
    src_p = jnp.concatenate([src, fill]).reshape(NW, NCH, GCH)
    dst_p = jnp.concatenate([dst, N + fill]).reshape(NW, NCH, GCH)
    e_feat_p = jnp.concatenate(
        [e_feat, jnp.zeros((npad, DE), jnp.float32)], axis=0)
    zeros_acc = jnp.zeros((N_ACC, D), jnp.float32)
    be1_2 = be1.reshape(1, DH)
    be2_2 = be2.reshape(1, D * D)
    cb_2 = conv_bias.reshape(1, D)
    bm_2 = bm.reshape(1, D)
    b0_2 = b0.reshape(1, D)
    wm1 = Wm[:D]
    wm2 = Wm[D:]
    P = jnp.kron(jnp.eye(D, dtype=jnp.float32), jnp.ones((1, D), jnp.float32))
    Wc = jnp.zeros((DH + D, 2 * D * D), jnp.float32)
    Wc = Wc.at[:DH, :D * D].set(We2).at[DH:, D * D:].set(P)

    n_feat_acc = jnp.concatenate(
        [n_feat, jnp.zeros((N_ACC - N, D), jnp.float32)], axis=0)
    sc_gather, sc_scatter = _sc_kernels()
    out = _prologue(n_feat_acc, W0, b0_2)
    for step in range(STEPS):
        h_src = sc_gather(out, src_p)
        msg = _msg_kernel(e_feat_p, h_src, We1, be1_2, Wc, be2_2)
        parts = sc_scatter(msg, dst_p, zeros_acc)
        if step == STEPS - 1:
            out = _final(parts, out, cb_2, wm1, wm2, bm_2, n_feat)
        else:
            out = _update(parts, out, cb_2, wm1, wm2, bm_2)
    return out


# speedup vs baseline: 1.2774x; 1.1938x over previous
import functools

import jax
import jax.numpy as jnp
from jax import lax
from jax.experimental import pallas as pl
from jax.experimental.pallas import tpu as pltpu
from jax.experimental.pallas import tpu_sc as plsc

N = 10000
E = 100000
D = 32
DE = 16
DH = 128
STEPS = 2

NC = 2
NS = 16
NW = NC * NS
GCH = 128
NCH = 25
EPW = NCH * GCH
E_PAD = NW * EPW
N_ACC = 10112
RPS = N_ACC // NS
TE = 1024


@functools.lru_cache(maxsize=1)
def _sc_kernels():
    mesh = plsc.VectorSubcoreMesh(core_axis_name="c", subcore_axis_name="s")

    @functools.partial(
        pl.kernel,
        mesh=mesh,
        out_type=jax.ShapeDtypeStruct((E_PAD, D), jnp.float32),
        scratch_types=[
            pltpu.VMEM((NCH, GCH), jnp.int32),
            pltpu.VMEM((EPW, D), jnp.float32),
            pltpu.VMEM_SHARED((N_ACC, D), jnp.float32),
            pltpu.SemaphoreType.DMA,
        ],
        compiler_params=pltpu.CompilerParams(use_tc_tiling_on_sc=False),
    )
    def _sc_gather(nodes_hbm, src_hbm, hsrc_hbm, idx_v, rows_v, nodes_sh, sem):
        c = lax.axis_index("c")
        s = lax.axis_index("s")
        wid = s * NC + c
        base = wid * EPW
        pltpu.sync_copy(nodes_hbm.at[pl.ds(s * RPS, RPS)],
                        nodes_sh.at[pl.ds(s * RPS, RPS)])
        pltpu.sync_copy(src_hbm.at[wid], idx_v)
        plsc.subcore_barrier()

        def _issue(j, carry):
            pltpu.async_copy(nodes_sh.at[idx_v.at[j]],
                             rows_v.at[pl.ds(j * GCH, GCH)], sem)
            return carry

        lax.fori_loop(0, NCH, _issue, 0)

        def _drain(j, carry):
            pltpu.make_async_copy(hsrc_hbm.at[pl.ds(0, GCH)],
                                  rows_v.at[pl.ds(0, GCH)], sem).wait()
            return carry

        lax.fori_loop(0, NCH, _drain, 0)
        pltpu.sync_copy(rows_v, hsrc_hbm.at[pl.ds(base, EPW)])

    @functools.partial(
        pl.kernel,
        mesh=mesh,
        out_type=jax.ShapeDtypeStruct((NC, N_ACC, D), jnp.float32),
        scratch_types=[
            pltpu.VMEM((NCH, GCH), jnp.int32),
            pltpu.VMEM((EPW, D), jnp.float32),
            pltpu.VMEM_SHARED((N_ACC, D), jnp.float32),
            pltpu.SemaphoreType.DMA,
        ],
        compiler_params=pltpu.CompilerParams(use_tc_tiling_on_sc=False),
    )
    def _sc_scatter(msg_hbm, dst_hbm, zeros_hbm, part_hbm,
                    idx_v, rows_v, acc_sh, sem):
        c = lax.axis_index("c")
        s = lax.axis_index("s")
        wid = c * NS + s
        base = wid * EPW

        pltpu.sync_copy(zeros_hbm.at[pl.ds(s * RPS, RPS)],
                        acc_sh.at[pl.ds(s * RPS, RPS)])
        plsc.subcore_barrier()

        pltpu.sync_copy(dst_hbm.at[wid], idx_v)
        pltpu.sync_copy(msg_hbm.at[pl.ds(base, EPW)], rows_v)

        def _scat(j, carry):
            pltpu.sync_copy(rows_v.at[pl.ds(j * GCH, GCH)],
                            acc_sh.at[idx_v.at[j]], add=True)
            return carry

        lax.fori_loop(0, NCH, _scat, 0)
        plsc.subcore_barrier()

        pltpu.sync_copy(acc_sh.at[pl.ds(s * RPS, RPS)],
                        part_hbm.at[c, pl.ds(s * RPS, RPS)])

    return _sc_gather, _sc_scatter


def _msg_body(ef_ref, hs_ref, we1_ref, be1_ref, wc_ref, be2_ref, msg_ref):
    henc = jnp.maximum(
        jnp.dot(ef_ref[...], we1_ref[...], preferred_element_type=jnp.float32)
        + be1_ref[...], 0.0)
    hs4 = hs_ref[...]
    hs = jnp.concatenate([hs4[:, q * D:(q + 1) * D] for q in range(4)],
                         axis=0)
    comb = jnp.dot(jnp.concatenate([henc, hs], axis=1), wc_ref[...],
                   preferred_element_type=jnp.float32)
    ewt = comb[:, :D * D] + be2_ref[...]
    hrep = comb[:, D * D:]
    prod = (hrep[:, :512] * ewt[:, :512] + hrep[:, 512:] * ewt[:, 512:])
    prod = prod[:, :256] + prod[:, 256:]
    prod = prod[:, :128] + prod[:, 128:]
    prod = prod[:, :64] + prod[:, 64:]
    msg = prod[:, :32] + prod[:, 32:]
    Q = TE // 4
    msg_ref[...] = jnp.concatenate([msg[q * Q:(q + 1) * Q, :]
                                    for q in range(4)], axis=1)


def _msg_kernel(e_feat_p, h_src4, We1, be1, Wc, be2):
    grid = (E_PAD // TE,)
    return pl.pallas_call(
        _msg_body,
        grid=grid,
        in_specs=[
            pl.BlockSpec((TE, DE), lambda i: (i, 0)),
            pl.BlockSpec((TE // 4, 4 * D), lambda i: (i, 0)),
            pl.BlockSpec((DE, DH), lambda i: (0, 0)),
            pl.BlockSpec((1, DH), lambda i: (0, 0)),
            pl.BlockSpec((DH + D, 2 * D * D), lambda i: (0, 0)),
            pl.BlockSpec((1, D * D), lambda i: (0, 0)),
        ],
        out_specs=pl.BlockSpec((TE // 4, 4 * D), lambda i: (i, 0)),
        out_shape=jax.ShapeDtypeStruct((E_PAD // 4, 4 * D), jnp.float32),
    )(e_feat_p, h_src4, We1, be1, Wc, be2)


def _prologue_body(nf_ref, w0_ref, b0_ref, out_ref):
    out_ref[...] = jnp.maximum(
        jnp.dot(nf_ref[...], w0_ref[...], preferred_element_type=jnp.float32)
        + b0_ref[...], 0.0)


def _prologue(n_feat_acc, W0, b0):
    return pl.pallas_call(
        _prologue_body,
        out_shape=jax.ShapeDtypeStruct((N_ACC, D), jnp.float32),
    )(n_feat_acc, W0, b0)


def _update_body(pa_ref, out_ref, cb_ref, wm1_ref, wm2_ref, bm_ref,
                 o_ref):
    out = out_ref[...]
    neigh = pa_ref[0] + pa_ref[1]
    m = jnp.maximum(neigh + out + cb_ref[...], 0.0)
    o_ref[...] = (jnp.dot(m, wm1_ref[...], preferred_element_type=jnp.float32)
                  + jnp.dot(out, wm2_ref[...],
                            preferred_element_type=jnp.float32)
                  + bm_ref[...])


def _final_body(pa_ref, out_ref, cb_ref, wm1_ref, wm2_ref, bm_ref,
                init_ref, o_ref):
    out = out_ref[:N, :]
    neigh = pa_ref[0, :N, :] + pa_ref[1, :N, :]
    m = jnp.maximum(neigh + out + cb_ref[...], 0.0)
    o_ref[...] = (jnp.dot(m, wm1_ref[...], preferred_element_type=jnp.float32)
                  + jnp.dot(out, wm2_ref[...],
                            preferred_element_type=jnp.float32)
                  + bm_ref[...] + init_ref[...])


def _update(pa, out, cb, wm1, wm2, bm):
    return pl.pallas_call(
        _update_body,
        out_shape=jax.ShapeDtypeStruct((N_ACC, D), jnp.float32),
    )(pa, out, cb, wm1, wm2, bm)


def _final(pa, out, cb, wm1, wm2, bm, init):
    return pl.pallas_call(
        _final_body,
        out_shape=jax.ShapeDtypeStruct((N, D), jnp.float32),
    )(pa, out, cb, wm1, wm2, bm, init)


def kernel(edge_index, n_feat, e_feat, W0, b0, We1, be1, We2, be2, conv_bias,
           Wm, bm):
    src = edge_index[0]
    dst = edge_index[1]
    npad = E_PAD - E
    fill = (jnp.arange(npad, dtype=jnp.int32) % (N_ACC - N))
    src_p = jnp.concatenate([src, fill]).reshape(NW, NCH, GCH)
    dst_p = jnp.concatenate([dst, N + fill]).reshape(NW, NCH, GCH)
    e_feat_p = jnp.concatenate(
        [e_feat, jnp.zeros((npad, DE), jnp.float32)], axis=0)
    zeros_acc = jnp.zeros((N_ACC, D), jnp.float32)
    be1_2 = be1.reshape(1, DH)
    be2_2 = be2.reshape(1, D * D)
    cb_2 = conv_bias.reshape(1, D)
    bm_2 = bm.reshape(1, D)
    b0_2 = b0.reshape(1, D)
    wm1 = Wm[:D]
    wm2 = Wm[D:]
    P = jnp.kron(jnp.eye(D, dtype=jnp.float32), jnp.ones((1, D), jnp.float32))
    Wc = jnp.zeros((DH + D, 2 * D * D), jnp.float32)
    Wc = Wc.at[:DH, :D * D].set(We2).at[DH:, D * D:].set(P)

    ef_perm = (e_feat_p.reshape(E_PAD // TE, TE // 4, 4, DE)
               .transpose(0, 2, 1, 3).reshape(E_PAD, DE))
    n_feat_acc = jnp.concatenate(
        [n_feat, jnp.zeros((N_ACC - N, D), jnp.float32)], axis=0)
    sc_gather, sc_scatter = _sc_kernels()
    out = _prologue(n_feat_acc, W0, b0_2)
    for step in range(STEPS):
        h_src = sc_gather(out, src_p)
        msg4 = _msg_kernel(ef_perm, h_src.reshape(E_PAD // 4, 4 * D),
                           We1, be1_2, Wc, be2_2)
        parts = sc_scatter(msg4.reshape(E_PAD, D), dst_p, zeros_acc)
        if step == STEPS - 1:
            out = _final(parts, out, cb_2, wm1, wm2, bm_2, n_feat)
        else:
            out = _update(parts, out, cb_2, wm1, wm2, bm_2)
    return out


# speedup vs baseline: 1.3731x; 1.0750x over previous
import functools

import jax
import jax.numpy as jnp
from jax import lax
from jax.experimental import pallas as pl
from jax.experimental.pallas import tpu as pltpu
from jax.experimental.pallas import tpu_sc as plsc

N = 10000
E = 100000
D = 32
DE = 16
DH = 128
STEPS = 2

NC = 2
NS = 16
NW = NC * NS
GCH = 128
NCH = 25
EPW = NCH * GCH
E_PAD = NW * EPW
N_ACC = 10112
RPS = N_ACC // NS
TE = 2048


@functools.lru_cache(maxsize=1)
def _sc_kernels():
    mesh = plsc.VectorSubcoreMesh(core_axis_name="c", subcore_axis_name="s")

    @functools.partial(
        pl.kernel,
        mesh=mesh,
        out_type=jax.ShapeDtypeStruct((E_PAD, D), jnp.float32),
        scratch_types=[
            pltpu.VMEM((NCH, GCH), jnp.int32),
            pltpu.VMEM((EPW, D), jnp.float32),
            pltpu.VMEM_SHARED((N_ACC, D), jnp.float32),
            pltpu.SemaphoreType.DMA,
        ],
        compiler_params=pltpu.CompilerParams(use_tc_tiling_on_sc=False),
    )
    def _sc_gather(nodes_hbm, src_hbm, hsrc_hbm, idx_v, rows_v, nodes_sh, sem):
        c = lax.axis_index("c")
        s = lax.axis_index("s")
        wid = s * NC + c
        base = wid * EPW
        pltpu.sync_copy(nodes_hbm.at[pl.ds(s * RPS, RPS)],
                        nodes_sh.at[pl.ds(s * RPS, RPS)])
        pltpu.sync_copy(src_hbm.at[wid], idx_v)
        plsc.subcore_barrier()

        def _issue(j, carry):
            pltpu.async_copy(nodes_sh.at[idx_v.at[j]],
                             rows_v.at[pl.ds(j * GCH, GCH)], sem)
            return carry

        lax.fori_loop(0, NCH, _issue, 0)

        def _drain(j, carry):
            pltpu.make_async_copy(hsrc_hbm.at[pl.ds(0, GCH)],
                                  rows_v.at[pl.ds(0, GCH)], sem).wait()
            return carry

        lax.fori_loop(0, NCH, _drain, 0)
        pltpu.sync_copy(rows_v, hsrc_hbm.at[pl.ds(base, EPW)])

    @functools.partial(
        pl.kernel,
        mesh=mesh,
        out_type=jax.ShapeDtypeStruct((NC, N_ACC, D), jnp.float32),
        scratch_types=[
            pltpu.VMEM((NCH, GCH), jnp.int32),
            pltpu.VMEM((EPW, D), jnp.float32),
            pltpu.VMEM_SHARED((N_ACC, D), jnp.float32),
            pltpu.SemaphoreType.DMA,
        ],
        compiler_params=pltpu.CompilerParams(use_tc_tiling_on_sc=False),
    )
    def _sc_scatter(msg_hbm, dst_hbm, zeros_hbm, part_hbm,
                    idx_v, rows_v, acc_sh, sem):
        c = lax.axis_index("c")
        s = lax.axis_index("s")
        wid = c * NS + s
        base = wid * EPW

        pltpu.sync_copy(zeros_hbm.at[pl.ds(s * RPS, RPS)],
                        acc_sh.at[pl.ds(s * RPS, RPS)])
        plsc.subcore_barrier()

        pltpu.sync_copy(dst_hbm.at[wid], idx_v)
        pltpu.sync_copy(msg_hbm.at[pl.ds(base, EPW)], rows_v)

        def _scat(j, carry):
            pltpu.sync_copy(rows_v.at[pl.ds(j * GCH, GCH)],
                            acc_sh.at[idx_v.at[j]], add=True)
            return carry

        lax.fori_loop(0, NCH, _scat, 0)
        plsc.subcore_barrier()

        pltpu.sync_copy(acc_sh.at[pl.ds(s * RPS, RPS)],
                        part_hbm.at[c, pl.ds(s * RPS, RPS)])

    return _sc_gather, _sc_scatter


def _msg_body(ef_ref, hs_ref, we1_ref, be1_ref, wc_ref, be2_ref, msg_ref):
    henc = jnp.maximum(
        jnp.dot(ef_ref[...], we1_ref[...], preferred_element_type=jnp.float32)
        + be1_ref[...], 0.0)
    hs4 = hs_ref[...]
    hs = jnp.concatenate([hs4[:, q * D:(q + 1) * D] for q in range(4)],
                         axis=0)
    comb = jnp.dot(jnp.concatenate([henc, hs], axis=1), wc_ref[...],
                   preferred_element_type=jnp.float32)
    ewt = comb[:, :D * D] + be2_ref[...]
    hrep = comb[:, D * D:]
    prod = (hrep[:, :512] * ewt[:, :512] + hrep[:, 512:] * ewt[:, 512:])
    prod = prod[:, :256] + prod[:, 256:]
    prod = prod[:, :128] + prod[:, 128:]
    prod = prod[:, :64] + prod[:, 64:]
    msg = prod[:, :32] + prod[:, 32:]
    Q = TE // 4
    msg_ref[...] = jnp.concatenate([msg[q * Q:(q + 1) * Q, :]
                                    for q in range(4)], axis=1)


def _msg_kernel(e_feat_p, h_src4, We1, be1, Wc, be2):
    grid = (E_PAD // TE,)
    return pl.pallas_call(
        _msg_body,
        grid=grid,
        in_specs=[
            pl.BlockSpec((TE, DE), lambda i: (i, 0)),
            pl.BlockSpec((TE // 4, 4 * D), lambda i: (i, 0)),
            pl.BlockSpec((DE, DH), lambda i: (0, 0)),
            pl.BlockSpec((1, DH), lambda i: (0, 0)),
            pl.BlockSpec((DH + D, 2 * D * D), lambda i: (0, 0)),
            pl.BlockSpec((1, D * D), lambda i: (0, 0)),
        ],
        out_specs=pl.BlockSpec((TE // 4, 4 * D), lambda i: (i, 0)),
        out_shape=jax.ShapeDtypeStruct((E_PAD // 4, 4 * D), jnp.float32),
    )(e_feat_p, h_src4, We1, be1, Wc, be2)


def _prologue_body(nf_ref, w0_ref, b0_ref, out_ref):
    out_ref[...] = jnp.maximum(
        jnp.dot(nf_ref[...], w0_ref[...], preferred_element_type=jnp.float32)
        + b0_ref[...], 0.0)


def _prologue(n_feat_acc, W0, b0):
    return pl.pallas_call(
        _prologue_body,
        out_shape=jax.ShapeDtypeStruct((N_ACC, D), jnp.float32),
    )(n_feat_acc, W0, b0)


def _update_body(pa_ref, out_ref, cb_ref, wm1_ref, wm2_ref, bm_ref,
                 o_ref):
    out = out_ref[...]
    neigh = pa_ref[0] + pa_ref[1]
    m = jnp.maximum(neigh + out + cb_ref[...], 0.0)
    o_ref[...] = (jnp.dot(m, wm1_ref[...], preferred_element_type=jnp.float32)
                  + jnp.dot(out, wm2_ref[...],
                            preferred_element_type=jnp.float32)
                  + bm_ref[...])


def _final_body(pa_ref, out_ref, cb_ref, wm1_ref, wm2_ref, bm_ref,
                init_ref, o_ref):
    out = out_ref[:N, :]
    neigh = pa_ref[0, :N, :] + pa_ref[1, :N, :]
    m = jnp.maximum(neigh + out + cb_ref[...], 0.0)
    o_ref[...] = (jnp.dot(m, wm1_ref[...], preferred_element_type=jnp.float32)
                  + jnp.dot(out, wm2_ref[...],
                            preferred_element_type=jnp.float32)
                  + bm_ref[...] + init_ref[...])


def _update(pa, out, cb, wm1, wm2, bm):
    return pl.pallas_call(
        _update_body,
        out_shape=jax.ShapeDtypeStruct((N_ACC, D), jnp.float32),
    )(pa, out, cb, wm1, wm2, bm)


def _final(pa, out, cb, wm1, wm2, bm, init):
    return pl.pallas_call(
        _final_body,
        out_shape=jax.ShapeDtypeStruct((N, D), jnp.float32),
    )(pa, out, cb, wm1, wm2, bm, init)


def kernel(edge_index, n_feat, e_feat, W0, b0, We1, be1, We2, be2, conv_bias,
           Wm, bm):
    src = edge_index[0]
    dst = edge_index[1]
    npad = E_PAD - E
    fill = (jnp.arange(npad, dtype=jnp.int32) % (N_ACC - N))
    src_p = jnp.concatenate([src, fill]).reshape(NW, NCH, GCH)
    dst_p = jnp.concatenate([dst, N + fill]).reshape(NW, NCH, GCH)
    e_feat_p = jnp.concatenate(
        [e_feat, jnp.zeros((npad, DE), jnp.float32)], axis=0)
    zeros_acc = jnp.zeros((N_ACC, D), jnp.float32)
    be1_2 = be1.reshape(1, DH)
    be2_2 = be2.reshape(1, D * D)
    cb_2 = conv_bias.reshape(1, D)
    bm_2 = bm.reshape(1, D)
    b0_2 = b0.reshape(1, D)
    wm1 = Wm[:D]
    wm2 = Wm[D:]
    P = jnp.kron(jnp.eye(D, dtype=jnp.float32), jnp.ones((1, D), jnp.float32))
    Wc = jnp.zeros((DH + D, 2 * D * D), jnp.float32)
    Wc = Wc.at[:DH, :D * D].set(We2).at[DH:, D * D:].set(P)

    ef_perm = (e_feat_p.reshape(E_PAD // TE, TE // 4, 4, DE)
               .transpose(0, 2, 1, 3).reshape(E_PAD, DE))
    n_feat_acc = jnp.concatenate(
        [n_feat, jnp.zeros((N_ACC - N, D), jnp.float32)], axis=0)
    sc_gather, sc_scatter = _sc_kernels()
    out = _prologue(n_feat_acc, W0, b0_2)
    for step in range(STEPS):
        h_src = sc_gather(out, src_p)
        msg4 = _msg_kernel(ef_perm, h_src.reshape(E_PAD // 4, 4 * D),
                           We1, be1_2, Wc, be2_2)
        parts = sc_scatter(msg4.reshape(E_PAD, D), dst_p, zeros_acc)
        if step == STEPS - 1:
            out = _final(parts, out, cb_2, wm1, wm2, bm_2, n_feat)
        else:
            out = _update(parts, out, cb_2, wm1, wm2, bm_2)
    return out


# speedup vs baseline: 1.3965x; 1.0170x over previous
import functools

import jax
import jax.numpy as jnp
from jax import lax
from jax.experimental import pallas as pl
from jax.experimental.pallas import tpu as pltpu
from jax.experimental.pallas import tpu_sc as plsc

N = 10000
E = 100000
D = 32
DE = 16
DH = 128
STEPS = 2

NC = 2
NS = 16
NW = NC * NS
GCH = 128
NCH = 25
EPW = NCH * GCH
E_PAD = NW * EPW
N_ACC = 10112
RPS = N_ACC // NS
TE = 4096


@functools.lru_cache(maxsize=1)
def _sc_kernels():
    mesh = plsc.VectorSubcoreMesh(core_axis_name="c", subcore_axis_name="s")

    @functools.partial(
        pl.kernel,
        mesh=mesh,
        out_type=jax.ShapeDtypeStruct((E_PAD, D), jnp.float32),
        scratch_types=[
            pltpu.VMEM((NCH, GCH), jnp.int32),
            pltpu.VMEM((EPW, D), jnp.float32),
            pltpu.VMEM_SHARED((N_ACC, D), jnp.float32),
            pltpu.SemaphoreType.DMA,
        ],
        compiler_params=pltpu.CompilerParams(use_tc_tiling_on_sc=False),
    )
    def _sc_gather(nodes_hbm, src_hbm, hsrc_hbm, idx_v, rows_v, nodes_sh, sem):
        c = lax.axis_index("c")
        s = lax.axis_index("s")
        wid = s * NC + c
        base = wid * EPW
        pltpu.sync_copy(nodes_hbm.at[pl.ds(s * RPS, RPS)],
                        nodes_sh.at[pl.ds(s * RPS, RPS)])
        pltpu.sync_copy(src_hbm.at[wid], idx_v)
        plsc.subcore_barrier()

        def _issue(j, carry):
            pltpu.async_copy(nodes_sh.at[idx_v.at[j]],
                             rows_v.at[pl.ds(j * GCH, GCH)], sem)
            return carry

        lax.fori_loop(0, NCH, _issue, 0)

        def _drain(j, carry):
            pltpu.make_async_copy(hsrc_hbm.at[pl.ds(0, GCH)],
                                  rows_v.at[pl.ds(0, GCH)], sem).wait()
            return carry

        lax.fori_loop(0, NCH, _drain, 0)
        pltpu.sync_copy(rows_v, hsrc_hbm.at[pl.ds(base, EPW)])

    @functools.partial(
        pl.kernel,
        mesh=mesh,
        out_type=jax.ShapeDtypeStruct((NC, N_ACC, D), jnp.float32),
        scratch_types=[
            pltpu.VMEM((NCH, GCH), jnp.int32),
            pltpu.VMEM((EPW, D), jnp.float32),
            pltpu.VMEM_SHARED((N_ACC, D), jnp.float32),
            pltpu.SemaphoreType.DMA,
        ],
        compiler_params=pltpu.CompilerParams(use_tc_tiling_on_sc=False),
    )
    def _sc_scatter(msg_hbm, dst_hbm, zeros_hbm, part_hbm,
                    idx_v, rows_v, acc_sh, sem):
        c = lax.axis_index("c")
        s = lax.axis_index("s")
        wid = c * NS + s
        base = wid * EPW

        pltpu.sync_copy(zeros_hbm.at[pl.ds(s * RPS, RPS)],
                        acc_sh.at[pl.ds(s * RPS, RPS)])
        plsc.subcore_barrier()

        pltpu.sync_copy(dst_hbm.at[wid], idx_v)
        pltpu.sync_copy(msg_hbm.at[pl.ds(base, EPW)], rows_v)

        def _scat(j, carry):
            pltpu.sync_copy(rows_v.at[pl.ds(j * GCH, GCH)],
                            acc_sh.at[idx_v.at[j]], add=True)
            return carry

        lax.fori_loop(0, NCH, _scat, 0)
        plsc.subcore_barrier()

        pltpu.sync_copy(acc_sh.at[pl.ds(s * RPS, RPS)],
                        part_hbm.at[c, pl.ds(s * RPS, RPS)])

    return _sc_gather, _sc_scatter


def _msg_body(ef_ref, hs_ref, we1_ref, be1_ref, wc_ref, be2_ref, msg_ref):
    henc = jnp.maximum(
        jnp.dot(ef_ref[...], we1_ref[...], preferred_element_type=jnp.float32)
        + be1_ref[...], 0.0)
    hs4 = hs_ref[...]
    hs = jnp.concatenate([hs4[:, q * D:(q + 1) * D] for q in range(4)],
                         axis=0)
    comb = jnp.dot(jnp.concatenate([henc, hs], axis=1), wc_ref[...],
                   preferred_element_type=jnp.float32)
    ewt = comb[:, :D * D] + be2_ref[...]
    hrep = comb[:, D * D:]
    prod = (hrep[:, :512] * ewt[:, :512] + hrep[:, 512:] * ewt[:, 512:])
    prod = prod[:, :256] + prod[:, 256:]
    prod = prod[:, :128] + prod[:, 128:]
    prod = prod[:, :64] + prod[:, 64:]
    msg = prod[:, :32] + prod[:, 32:]
    Q = TE // 4
    msg_ref[...] = jnp.concatenate([msg[q * Q:(q + 1) * Q, :]
                                    for q in range(4)], axis=1)


def _msg_kernel(e_feat_p, h_src4, We1, be1, Wc, be2):
    grid = (E_PAD // TE,)
    return pl.pallas_call(
        _msg_body,
        grid=grid,
        in_specs=[
            pl.BlockSpec((TE, DE), lambda i: (i, 0)),
            pl.BlockSpec((TE // 4, 4 * D), lambda i: (i, 0)),
            pl.BlockSpec((DE, DH), lambda i: (0, 0)),
            pl.BlockSpec((1, DH), lambda i: (0, 0)),
            pl.BlockSpec((DH + D, 2 * D * D), lambda i: (0, 0)),
            pl.BlockSpec((1, D * D), lambda i: (0, 0)),
        ],
        out_specs=pl.BlockSpec((TE // 4, 4 * D), lambda i: (i, 0)),
        out_shape=jax.ShapeDtypeStruct((E_PAD // 4, 4 * D), jnp.float32),
    )(e_feat_p, h_src4, We1, be1, Wc, be2)


def _prologue_body(nf_ref, w0_ref, b0_ref, out_ref):
    out_ref[...] = jnp.maximum(
        jnp.dot(nf_ref[...], w0_ref[...], preferred_element_type=jnp.float32)
        + b0_ref[...], 0.0)


def _prologue(n_feat_acc, W0, b0):
    return pl.pallas_call(
        _prologue_body,
        out_shape=jax.ShapeDtypeStruct((N_ACC, D), jnp.float32),
    )(n_feat_acc, W0, b0)


def _update_body(pa_ref, out_ref, cb_ref, wm1_ref, wm2_ref, bm_ref,
                 o_ref):
    out = out_ref[...]
    neigh = pa_ref[0] + pa_ref[1]
    m = jnp.maximum(neigh + out + cb_ref[...], 0.0)
    o_ref[...] = (jnp.dot(m, wm1_ref[...], preferred_element_type=jnp.float32)
                  + jnp.dot(out, wm2_ref[...],
                            preferred_element_type=jnp.float32)
                  + bm_ref[...])


def _final_body(pa_ref, out_ref, cb_ref, wm1_ref, wm2_ref, bm_ref,
                init_ref, o_ref):
    out = out_ref[:N, :]
    neigh = pa_ref[0, :N, :] + pa_ref[1, :N, :]
    m = jnp.maximum(neigh + out + cb_ref[...], 0.0)
    o_ref[...] = (jnp.dot(m, wm1_ref[...], preferred_element_type=jnp.float32)
                  + jnp.dot(out, wm2_ref[...],
                            preferred_element_type=jnp.float32)
                  + bm_ref[...] + init_ref[...])


def _update(pa, out, cb, wm1, wm2, bm):
    return pl.pallas_call(
        _update_body,
        out_shape=jax.ShapeDtypeStruct((N_ACC, D), jnp.float32),
    )(pa, out, cb, wm1, wm2, bm)


def _final(pa, out, cb, wm1, wm2, bm, init):
    return pl.pallas_call(
        _final_body,
        out_shape=jax.ShapeDtypeStruct((N, D), jnp.float32),
    )(pa, out, cb, wm1, wm2, bm, init)


def kernel(edge_index, n_feat, e_feat, W0, b0, We1, be1, We2, be2, conv_bias,
           Wm, bm):
    src = edge_index[0]
    dst = edge_index[1]
    npad = E_PAD - E
    fill = (jnp.arange(npad, dtype=jnp.int32) % (N_ACC - N))
    src_p = jnp.concatenate([src, fill]).reshape(NW, NCH, GCH)
    dst_p = jnp.concatenate([dst, N + fill]).reshape(NW, NCH, GCH)
    e_feat_p = jnp.concatenate(
        [e_feat, jnp.zeros((npad, DE), jnp.float32)], axis=0)
    zeros_acc = jnp.zeros((N_ACC, D), jnp.float32)
    be1_2 = be1.reshape(1, DH)
    be2_2 = be2.reshape(1, D * D)
    cb_2 = conv_bias.reshape(1, D)
    bm_2 = bm.reshape(1, D)
    b0_2 = b0.reshape(1, D)
    wm1 = Wm[:D]
    wm2 = Wm[D:]
    P = jnp.kron(jnp.eye(D, dtype=jnp.float32), jnp.ones((1, D), jnp.float32))
    Wc = jnp.zeros((DH + D, 2 * D * D), jnp.float32)
    Wc = Wc.at[:DH, :D * D].set(We2).at[DH:, D * D:].set(P)

    ef_perm = (e_feat_p.reshape(E_PAD // TE, TE // 4, 4, DE)
               .transpose(0, 2, 1, 3).reshape(E_PAD, DE))
    n_feat_acc = jnp.concatenate(
        [n_feat, jnp.zeros((N_ACC - N, D), jnp.float32)], axis=0)
    sc_gather, sc_scatter = _sc_kernels()
    out = _prologue(n_feat_acc, W0, b0_2)
    for step in range(STEPS):
        h_src = sc_gather(out, src_p)
        msg4 = _msg_kernel(ef_perm, h_src.reshape(E_PAD // 4, 4 * D),
                           We1, be1_2, Wc, be2_2)
        parts = sc_scatter(msg4.reshape(E_PAD, D), dst_p, zeros_acc)
        if step == STEPS - 1:
            out = _final(parts, out, cb_2, wm1, wm2, bm_2, n_feat)
        else:
            out = _update(parts, out, cb_2, wm1, wm2, bm_2)
    return out
